# baseline (speedup 1.0000x reference)
import jax
import jax.numpy as jnp
from jax.experimental import pallas as pl
from jax.experimental.pallas import tpu as pltpu

_SCALE = 0.125
_NUM_BUCKETS = 32

_NMIN = (
    0, 1, 2, 3, 4, 5, 6, 7, 8, 9, 10, 11, 12, 13, 14, 15,
    16, 19, 21, 24, 27, 31, 35, 40, 46, 52, 59, 67, 77, 87, 99, 113,
)

_N = 4096
_H = 256
_S = _N // _H
_W = _N + (_S - 1) * _H


def _bias_kernel(table_ref, out_ref, buf, sems):
    t0 = table_ref[0, 0] * _SCALE

    lane = jax.lax.broadcasted_iota(jnp.int32, (8, 128), 1)
    w = jnp.full((8, 128), t0, dtype=jnp.float32)
    for b in range(1, _NUM_BUCKETS):
        w = jnp.where(lane >= _NMIN[b], table_ref[b, 0] * _SCALE, w)
    w_b = jnp.broadcast_to(w[0:1, :], (_H, 128))

    t31 = table_ref[_NUM_BUCKETS - 1, 0] * _SCALE

    def gather_chunk(u0):
        row = jax.lax.broadcasted_iota(jnp.int32, (_H, _H), 0)
        col = jax.lax.broadcasted_iota(jnp.int32, (_H, _H), 1)
        idx = jnp.clip(row - (col + u0) + (_S - 1) * _H, 0, 127)
        return jnp.take_along_axis(w_b, idx, axis=1)

    copies = []

    def fire(r):
        c = pltpu.make_async_copy(
            buf.at[:, pl.ds((_S - 1 - r) * _H, _N)],
            out_ref.at[pl.ds(r * _H, _H)],
            sems.at[r],
        )
        c.start()
        copies.append(c)

    buf[:, pl.ds((_S - 1) * _H, _H)] = gather_chunk((_S - 1) * _H)
    buf[:, pl.ds(_N, _W - _N)] = jnp.full((_H, _W - _N), t0, dtype=jnp.float32)
    fire(0)
    buf[:, pl.ds((_S - 2) * _H, _H)] = gather_chunk((_S - 2) * _H)
    fire(1)
    c31 = jnp.full((_H, _H), t31, dtype=jnp.float32)
    for r in range(2, _S):
        buf[:, pl.ds((_S - 1 - r) * _H, _H)] = c31
        fire(r)

    for c in copies:
        c.wait()


@jax.jit
def kernel(x, table):
    i, j = x.shape[-2], x.shape[-1]
    return pl.pallas_call(
        _bias_kernel,
        in_specs=[pl.BlockSpec(memory_space=pltpu.SMEM)],
        out_specs=pl.BlockSpec(memory_space=pl.ANY),
        out_shape=jax.ShapeDtypeStruct((i, j), jnp.float32),
        scratch_shapes=[
            pltpu.VMEM((_H, _W), jnp.float32),
            pltpu.SemaphoreType.DMA((_S,)),
        ],
    )(table)


# speedup vs baseline: 1.0387x; 1.0387x over previous
import jax
import jax.numpy as jnp
from jax.experimental import pallas as pl
from jax.experimental.pallas import tpu as pltpu

_SCALE = 0.125
_NUM_BUCKETS = 32

_NMIN = (
    0, 1, 2, 3, 4, 5, 6, 7, 8, 9, 10, 11, 12, 13, 14, 15,
    16, 19, 21, 24, 27, 31, 35, 40, 46, 52, 59, 67, 77, 87, 99, 113,
)

_N = 4096
_H = 128
_S = _N // _H
_W = _N + (_S - 1) * _H


def _bias_kernel(table_ref, out_ref, buf, sems):
    t0 = table_ref[0, 0] * _SCALE

    lane = jax.lax.broadcasted_iota(jnp.int32, (8, 128), 1)
    w = jnp.full((8, 128), t0, dtype=jnp.float32)
    for b in range(1, _NUM_BUCKETS):
        w = jnp.where(lane >= _NMIN[b], table_ref[b, 0] * _SCALE, w)
    w_b = jnp.broadcast_to(w[0:1, :], (_H, 128))

    t31 = table_ref[_NUM_BUCKETS - 1, 0] * _SCALE

    def gather_chunk(u0):
        row = jax.lax.broadcasted_iota(jnp.int32, (_H, _H), 0)
        col = jax.lax.broadcasted_iota(jnp.int32, (_H, _H), 1)
        idx = jnp.clip(row - (col + u0) + (_S - 1) * _H, 0, 127)
        return jnp.take_along_axis(w_b, idx, axis=1)

    copies = []

    def fire(r):
        c = pltpu.make_async_copy(
            buf.at[:, pl.ds((_S - 1 - r) * _H, _N)],
            out_ref.at[pl.ds(r * _H, _H)],
            sems.at[r],
        )
        c.start()
        copies.append(c)

    buf[:, pl.ds((_S - 1) * _H, _H)] = gather_chunk((_S - 1) * _H)
    buf[:, pl.ds(_N, _W - _N)] = jnp.full((_H, _W - _N), t0, dtype=jnp.float32)
    fire(0)
    buf[:, pl.ds((_S - 2) * _H, _H)] = gather_chunk((_S - 2) * _H)
    fire(1)
    c31 = jnp.full((_H, _H), t31, dtype=jnp.float32)
    for r in range(2, _S):
        buf[:, pl.ds((_S - 1 - r) * _H, _H)] = c31
        fire(r)

    for c in copies:
        c.wait()


@jax.jit
def kernel(x, table):
    i, j = x.shape[-2], x.shape[-1]
    return pl.pallas_call(
        _bias_kernel,
        in_specs=[pl.BlockSpec(memory_space=pltpu.SMEM)],
        out_specs=pl.BlockSpec(memory_space=pl.ANY),
        out_shape=jax.ShapeDtypeStruct((i, j), jnp.float32),
        scratch_shapes=[
            pltpu.VMEM((_H, _W), jnp.float32),
            pltpu.SemaphoreType.DMA((_S,)),
        ],
    )(table)
